# baseline (speedup 1.0000x reference)
import functools

import jax
import jax.numpy as jnp
from jax import lax
from jax.experimental import pallas as pl
from jax.experimental.pallas import tpu as pltpu
from jax.experimental.pallas import tpu_sc as plsc

EPS = 1e-05
L = 16


def _rsqrt_newton(x):
    i = plsc.bitcast(x, jnp.int32)
    i = 0x5F3759DF - lax.shift_right_logical(i, 1)
    y = plsc.bitcast(i, jnp.float32)
    half_x = x * 0.5
    for _ in range(3):
        y = y * (1.5 - half_x * y * y)
    return y


def _make_sc_kernel(B, V, D, NC, NW, C):
    n_chunks_per_w = (B // NW) // C
    n_slices = D // L
    mesh = plsc.VectorSubcoreMesh(core_axis_name="c", subcore_axis_name="s")

    @functools.partial(
        pl.kernel,
        out_type=jax.ShapeDtypeStruct((B, D), jnp.float32),
        mesh=mesh,
        compiler_params=pltpu.CompilerParams(needs_layout_passes=False),
        scratch_types=[
            pltpu.VMEM((n_chunks_per_w, C), jnp.int32),
            pltpu.VMEM((C, D), jnp.float32),
            pltpu.VMEM((D,), jnp.float32),
            pltpu.VMEM((D,), jnp.float32),
            pltpu.SemaphoreType.DMA,
        ],
    )
    def sc_kernel(ids_hbm, table_hbm, gamma_hbm, beta_hbm, out_hbm,
                  idx_v, buf, gam_v, bet_v, gsem):
        wid = lax.axis_index("s") * NC + lax.axis_index("c")
        pltpu.sync_copy(ids_hbm.at[wid], idx_v)
        pltpu.sync_copy(gamma_hbm, gam_v)
        pltpu.sync_copy(beta_hbm, bet_v)
        base = wid * (n_chunks_per_w * C)

        def chunk_body(ci, carry):
            pltpu.async_copy(table_hbm.at[idx_v.at[ci]], buf, gsem).wait()

            def row_body(r, rcarry):
                s = jnp.zeros((L,), jnp.float32)
                q = jnp.zeros((L,), jnp.float32)
                for j in range(n_slices):
                    x = buf[r, pl.ds(j * L, L)]
                    s = s + x
                    q = q + x * x
                mean = jnp.sum(s) * (1.0 / D)
                meansq = jnp.sum(q) * (1.0 / D)
                var = meansq - mean * mean
                rstd = _rsqrt_newton(jnp.full((L,), var + EPS, jnp.float32))
                mean_v = jnp.full((L,), mean, jnp.float32)
                for j in range(n_slices):
                    g = gam_v[pl.ds(j * L, L)]
                    bt = bet_v[pl.ds(j * L, L)]
                    x = buf[r, pl.ds(j * L, L)]
                    buf[r, pl.ds(j * L, L)] = (x - mean_v) * rstd * g + bt
                return rcarry

            lax.fori_loop(0, C, row_body, 0, unroll=False)
            pltpu.sync_copy(buf, out_hbm.at[pl.ds(base + ci * C, C)])
            return carry

        lax.fori_loop(0, n_chunks_per_w, chunk_body, 0, unroll=False)

    return sc_kernel


def kernel(input_ids, word_embeddings, ln_gamma, ln_beta):
    orig_shape = input_ids.shape
    V, D = word_embeddings.shape
    B = input_ids.size
    info = plsc.get_sparse_core_info()
    NC, NS = info.num_cores, info.num_subcores
    NW = NC * NS
    C = 32

    ids = input_ids.reshape(NW, (B // NW) // C, C).astype(jnp.int32)
    sc = _make_sc_kernel(B, V, D, NC, NW, C)
    out = sc(ids, word_embeddings, ln_gamma, ln_beta)
    return out.reshape(-1, orig_shape[-1], D)
